# baseline (speedup 1.0000x reference)
import jax
import jax.numpy as jnp
from jax import lax
from jax.experimental import pallas as pl
from jax.experimental.pallas import tpu as pltpu
from jax.experimental.pallas import tpu_sc as plsc

N_NODES = 10000
N_EDGES = 320000
NC = 2
NS = 16
NW = NC * NS

GCH = 80
GNCH = (N_EDGES // NW) // GCH
GRND = 5
GSTR = GNCH // GRND
GROWS = GSTR * GCH

HEDGES = N_EDGES // 2
SC_EDGES = HEDGES // 16
SCH = 400
SNCH = SC_EDGES // SCH
NHALF = N_NODES // 2

_GATHER_DNUMS = lax.GatherDimensionNumbers(
    offset_dims=(), collapsed_slice_dims=(0,), start_index_map=(0,))


def _lane_bcast(v, i):
    idx = jnp.full((16,), i, jnp.int32)
    return lax.gather(v, idx[:, None], dimension_numbers=_GATHER_DNUMS,
                      slice_sizes=(1,),
                      mode=lax.GatherScatterMode.PROMISE_IN_BOUNDS)


def _node_pre_body(x8_ref, kw_ref, ku_ref, bu_ref, xw8_ref, xu8_ref):
    x8 = x8_ref[...]
    xw8_ref[...] = jnp.dot(x8, kw_ref[...], preferred_element_type=jnp.float32)
    xu8_ref[...] = jnp.dot(x8, ku_ref[...], preferred_element_type=jnp.float32) + bu_ref[...]


def _mid_body(ea8_ref, g8_ref, k1_ref, b1_ref, k2_ref, b2_ref, m8_ref):
    pre = jnp.dot(ea8_ref[...], k1_ref[...], preferred_element_type=jnp.float32)
    h = jnp.maximum(pre + b1_ref[...] + g8_ref[...], 0.0)
    m8_ref[...] = jnp.dot(h, k2_ref[...], preferred_element_type=jnp.float32) + b2_ref[...]


def _post_body(p8a_ref, p8b_ref, xu8_ref, k1b_ref, k2_ref, b2_ref, out8_ref):
    r8 = jnp.maximum(jnp.max(p8a_ref[...], axis=0), jnp.max(p8b_ref[...], axis=0))
    r8 = jnp.where(jnp.isfinite(r8), r8, 0.0)
    u8 = jnp.maximum(
        xu8_ref[...] + jnp.dot(r8, k1b_ref[...], preferred_element_type=jnp.float32),
        0.0,
    )
    out8_ref[...] = jnp.dot(u8, k2_ref[...], preferred_element_type=jnp.float32) + b2_ref[...]


def _node_pre(x8, kw, ku, bu):
    blk = 1250
    return pl.pallas_call(
        _node_pre_body,
        grid=(N_NODES // 8 // blk,),
        in_specs=[
            pl.BlockSpec((blk, 1024), lambda i: (i, 0)),
            pl.BlockSpec((1024, 128), lambda i: (0, 0)),
            pl.BlockSpec((1024, 128), lambda i: (0, 0)),
            pl.BlockSpec((1, 128), lambda i: (0, 0)),
        ],
        out_specs=[
            pl.BlockSpec((blk, 128), lambda i: (i, 0)),
            pl.BlockSpec((blk, 128), lambda i: (i, 0)),
        ],
        out_shape=[
            jax.ShapeDtypeStruct((N_NODES // 8, 128), jnp.float32),
            jax.ShapeDtypeStruct((N_NODES // 8, 128), jnp.float32),
        ],
    )(x8, kw, ku, bu)


def _mid(ea8, g8, g8_off, k1, b1, k2, b2):
    blk = 2000
    ne8 = ea8.shape[0]
    return pl.pallas_call(
        _mid_body,
        grid=(ne8 // blk,),
        in_specs=[
            pl.BlockSpec((blk, 128), lambda i: (i, 0)),
            pl.BlockSpec((blk, 128), lambda i: (i + g8_off, 0)),
            pl.BlockSpec((128, 128), lambda i: (0, 0)),
            pl.BlockSpec((1, 128), lambda i: (0, 0)),
            pl.BlockSpec((128, 128), lambda i: (0, 0)),
            pl.BlockSpec((1, 128), lambda i: (0, 0)),
        ],
        out_specs=pl.BlockSpec((blk, 128), lambda i: (i, 0)),
        out_shape=jax.ShapeDtypeStruct((ne8, 128), jnp.float32),
    )(ea8, g8, k1, b1, k2, b2)


def _post(p8a, p8b, xu8, k1b, k2, b2):
    blk = 1250
    return pl.pallas_call(
        _post_body,
        grid=(N_NODES // 8 // blk,),
        in_specs=[
            pl.BlockSpec((16, blk, 128), lambda i: (0, i, 0)),
            pl.BlockSpec((16, blk, 128), lambda i: (0, i, 0)),
            pl.BlockSpec((blk, 128), lambda i: (i, 0)),
            pl.BlockSpec((128, 128), lambda i: (0, 0)),
            pl.BlockSpec((128, 1024), lambda i: (0, 0)),
            pl.BlockSpec((1, 1024), lambda i: (0, 0)),
        ],
        out_specs=pl.BlockSpec((blk, 1024), lambda i: (i, 0)),
        out_shape=jax.ShapeDtypeStruct((N_NODES // 8, 1024), jnp.float32),
    )(p8a, p8b, xu8, k1b, k2, b2)


def _sc_gather_body(xw_hbm, src_hbm, g_hbm, idx_v, rows_v, sem):
    wid = lax.axis_index("s") * NC + lax.axis_index("c")
    e0 = wid * (GNCH * GCH)
    pltpu.sync_copy(src_hbm.at[pl.ds(e0, GNCH * GCH)], idx_v)

    def body(r, carry):
        for k in range(GSTR):
            pltpu.async_copy(
                xw_hbm.at[idx_v.at[pl.ds(r * GROWS + k * GCH, GCH)]],
                rows_v.at[pl.ds(k * GCH, GCH)], sem)
        pltpu.make_async_copy(xw_hbm.at[pl.ds(0, GROWS)], rows_v, sem).wait()
        pltpu.sync_copy(rows_v, g_hbm.at[pl.ds(e0 + r * GROWS, GROWS)])
        return carry

    lax.fori_loop(0, GRND, body, 0)


def _sc_gather(xw, src):
    return pl.kernel(
        _sc_gather_body,
        out_type=jax.ShapeDtypeStruct((N_EDGES, 16), jnp.float32),
        mesh=plsc.VectorSubcoreMesh(core_axis_name="c", subcore_axis_name="s"),
        compiler_params=pltpu.CompilerParams(use_tc_tiling_on_sc=False, needs_layout_passes=False),
        scratch_types=[
            pltpu.VMEM((GNCH * GCH,), jnp.int32),
            pltpu.VMEM((GROWS, 16), jnp.float32),
            pltpu.SemaphoreType.DMA,
        ],
    )(xw, src)


def _sc_scatter_body(m_hbm, dst_hbm, part_hbm, dst_v, m_v, r_v, sem0, sem1):
    wid = lax.axis_index("s") * NC + lax.axis_index("c")
    chunk = wid // 2
    nh = wid % 2
    base_node = nh * NHALF
    neg = jnp.full((16,), -jnp.inf, jnp.float32)
    iota = lax.broadcasted_iota(jnp.int32, (16,), 0)

    def fire(j, b, sem):
        ebase = chunk * SC_EDGES + j * SCH
        pltpu.async_copy(dst_hbm.at[pl.ds(ebase, SCH)], dst_v.at[b], sem)
        pltpu.async_copy(m_hbm.at[pl.ds(ebase * 16, SCH * 16)], m_v.at[b], sem)

    def drain(b, sem):
        pltpu.make_async_copy(dst_hbm.at[pl.ds(0, SCH)], dst_v.at[b], sem).wait()
        pltpu.make_async_copy(m_hbm.at[pl.ds(0, SCH * 16)], m_v.at[b], sem).wait()

    def process(b):
        def group(g, c2):
            d16 = dst_v[b, pl.ds(g * 16, 16)]
            for i2 in range(16):
                d_b = _lane_bcast(d16, i2)
                local = d_b - base_node
                inb = (local >= 0) & (local < NHALF)
                rowi = jnp.where(inb, local, NHALF)
                fidx = rowi * 16 + iota
                mrow = m_v[b, pl.ds((g * 16 + i2) * 16, 16)]
                cur = plsc.load_gather(r_v, [fidx])
                plsc.store_scatter(r_v, [fidx], jnp.maximum(cur, mrow))
            return c2

        lax.fori_loop(0, SCH // 16, group, 0)

    sems = (sem0, sem1)
    fire(0, 0, sems[0])

    def init(i, carry):
        r_v[pl.ds(i * 16, 16)] = neg
        return carry

    lax.fori_loop(0, NHALF + 1, init, 0)

    def do_pair(j0, carry):
        for b in range(2):
            j = j0 * 2 + b
            drain(b, sems[b])

            @pl.when(j < SNCH - 1)
            def _():
                fire(j + 1, 1 - b, sems[1 - b])

            process(b)
        return carry

    lax.fori_loop(0, (SNCH - 1) // 2, do_pair, 0)
    drain(0, sems[0])
    process(0)
    pltpu.sync_copy(r_v.at[pl.ds(0, NHALF * 16)], part_hbm.at[wid])


def _sc_scatter(m_flat, dst):
    return pl.kernel(
        _sc_scatter_body,
        out_type=jax.ShapeDtypeStruct((NW, NHALF * 16), jnp.float32),
        mesh=plsc.VectorSubcoreMesh(core_axis_name="c", subcore_axis_name="s"),
        compiler_params=pltpu.CompilerParams(use_tc_tiling_on_sc=False, needs_layout_passes=False),
        scratch_types=[
            pltpu.VMEM((2, SCH), jnp.int32),
            pltpu.VMEM((2, SCH * 16), jnp.float32),
            pltpu.VMEM(((NHALF + 1) * 16,), jnp.float32),
            pltpu.SemaphoreType.DMA,
            pltpu.SemaphoreType.DMA,
        ],
    )(m_flat, dst)


def kernel(x, edge_index, edge_attr, W_msg1, b_msg1, W_msg2, b_msg2,
           W_udt1, b_udt1, W_udt2, b_udt2):
    src = edge_index[0]
    dst = edge_index[1]
    eye8 = jnp.eye(8, dtype=jnp.float32)

    x8 = x.reshape(N_NODES // 8, 1024)
    kw = jnp.kron(eye8, W_msg1[16:])
    ku = jnp.kron(eye8, W_udt1[:128])
    k2 = jnp.kron(eye8, W_msg2)
    k1b = jnp.kron(eye8, W_udt1[128:])
    k2u = jnp.kron(eye8, W_udt2)
    bu8 = jnp.tile(b_udt1, 8).reshape(1, 128)
    b28 = jnp.tile(b_msg2, 8).reshape(1, 128)
    b2u8 = jnp.tile(b_udt2, 8).reshape(1, 1024)

    xw8, xu8 = _node_pre(x8, kw, ku, bu8)

    k1 = jnp.kron(eye8, W_msg1[:16])
    b18 = jnp.tile(b_msg1, 8).reshape(1, 128)

    g = _sc_gather(xw8.reshape(N_NODES, 16), src)
    g8 = g.reshape(N_EDGES // 8, 128)
    he8 = HEDGES // 8
    ea8_0 = edge_attr[:HEDGES].reshape(he8, 128)
    ea8_1 = edge_attr[HEDGES:].reshape(he8, 128)
    m8_0 = _mid(ea8_0, g8, 0, k1, b18, k2, b28)
    parts0 = _sc_scatter(m8_0.reshape(-1), dst[:HEDGES])
    m8_1 = _mid(ea8_1, g8, he8 // 2000, k1, b18, k2, b28)
    parts1 = _sc_scatter(m8_1.reshape(-1), dst[HEDGES:])
    p8a = parts0.reshape(16, N_NODES // 8, 128)
    p8b = parts1.reshape(16, N_NODES // 8, 128)

    out8 = _post(p8a, p8b, xu8, k1b, k2u, b2u8)
    return out8.reshape(N_NODES, 128)


# speedup vs baseline: 1.1882x; 1.1882x over previous
import jax
import jax.numpy as jnp
from jax import lax
from jax.experimental import pallas as pl
from jax.experimental.pallas import tpu as pltpu
from jax.experimental.pallas import tpu_sc as plsc

N_NODES = 10000
N_EDGES = 320000
NC = 2
NS = 16
NW = NC * NS

GCH = 80
GNCH = (N_EDGES // NW) // GCH
GRND = 5
GSTR = GNCH // GRND
GROWS = GSTR * GCH

SC_EDGES = N_EDGES // 16
SCH = 800
SNCH = SC_EDGES // SCH
NHALF = N_NODES // 2

_GATHER_DNUMS = lax.GatherDimensionNumbers(
    offset_dims=(), collapsed_slice_dims=(0,), start_index_map=(0,))


def _lane_bcast(v, i):
    idx = jnp.full((16,), i, jnp.int32)
    return lax.gather(v, idx[:, None], dimension_numbers=_GATHER_DNUMS,
                      slice_sizes=(1,),
                      mode=lax.GatherScatterMode.PROMISE_IN_BOUNDS)


def _node_pre_body(x8_ref, kw_ref, ku_ref, bu_ref, xw8_ref, xu8_ref):
    x8 = x8_ref[...]
    xw8_ref[...] = jnp.dot(x8, kw_ref[...], preferred_element_type=jnp.float32)
    xu8_ref[...] = jnp.dot(x8, ku_ref[...], preferred_element_type=jnp.float32) + bu_ref[...]


def _mid_body(ea8_ref, g8_ref, k1_ref, b1_ref, k2_ref, b2_ref, m8_ref):
    pre = jnp.dot(ea8_ref[...], k1_ref[...], preferred_element_type=jnp.float32)
    h = jnp.maximum(pre + b1_ref[...] + g8_ref[...], 0.0)
    m8_ref[...] = jnp.dot(h, k2_ref[...], preferred_element_type=jnp.float32) + b2_ref[...]


def _post_body(p8_ref, xu8_ref, k1b_ref, k2_ref, b2_ref, out8_ref):
    r8 = jnp.max(p8_ref[...], axis=0)
    r8 = jnp.where(jnp.isfinite(r8), r8, 0.0)
    u8 = jnp.maximum(
        xu8_ref[...] + jnp.dot(r8, k1b_ref[...], preferred_element_type=jnp.float32),
        0.0,
    )
    out8_ref[...] = jnp.dot(u8, k2_ref[...], preferred_element_type=jnp.float32) + b2_ref[...]


def _node_pre(x8, kw, ku, bu):
    blk = 1250
    return pl.pallas_call(
        _node_pre_body,
        grid=(N_NODES // 8 // blk,),
        in_specs=[
            pl.BlockSpec((blk, 1024), lambda i: (i, 0)),
            pl.BlockSpec((1024, 128), lambda i: (0, 0)),
            pl.BlockSpec((1024, 128), lambda i: (0, 0)),
            pl.BlockSpec((1, 128), lambda i: (0, 0)),
        ],
        out_specs=[
            pl.BlockSpec((blk, 128), lambda i: (i, 0)),
            pl.BlockSpec((blk, 128), lambda i: (i, 0)),
        ],
        out_shape=[
            jax.ShapeDtypeStruct((N_NODES // 8, 128), jnp.float32),
            jax.ShapeDtypeStruct((N_NODES // 8, 128), jnp.float32),
        ],
    )(x8, kw, ku, bu)


def _mid(ea8, g8, g8_off, k1, b1, k2, b2):
    blk = 2000
    ne8 = ea8.shape[0]
    return pl.pallas_call(
        _mid_body,
        grid=(ne8 // blk,),
        in_specs=[
            pl.BlockSpec((blk, 128), lambda i: (i, 0)),
            pl.BlockSpec((blk, 128), lambda i: (i + g8_off, 0)),
            pl.BlockSpec((128, 128), lambda i: (0, 0)),
            pl.BlockSpec((1, 128), lambda i: (0, 0)),
            pl.BlockSpec((128, 128), lambda i: (0, 0)),
            pl.BlockSpec((1, 128), lambda i: (0, 0)),
        ],
        out_specs=pl.BlockSpec((blk, 128), lambda i: (i, 0)),
        out_shape=jax.ShapeDtypeStruct((ne8, 128), jnp.float32),
    )(ea8, g8, k1, b1, k2, b2)


def _post(p8, xu8, k1b, k2, b2):
    blk = 1250
    return pl.pallas_call(
        _post_body,
        grid=(N_NODES // 8 // blk,),
        in_specs=[
            pl.BlockSpec((16, blk, 128), lambda i: (0, i, 0)),
            pl.BlockSpec((blk, 128), lambda i: (i, 0)),
            pl.BlockSpec((128, 128), lambda i: (0, 0)),
            pl.BlockSpec((128, 1024), lambda i: (0, 0)),
            pl.BlockSpec((1, 1024), lambda i: (0, 0)),
        ],
        out_specs=pl.BlockSpec((blk, 1024), lambda i: (i, 0)),
        out_shape=jax.ShapeDtypeStruct((N_NODES // 8, 1024), jnp.float32),
    )(p8, xu8, k1b, k2, b2)


def _sc_gather_body(xw_hbm, src_hbm, g_hbm, idx_v, rows_v, sem):
    wid = lax.axis_index("s") * NC + lax.axis_index("c")
    e0 = wid * (GNCH * GCH)
    pltpu.sync_copy(src_hbm.at[pl.ds(e0, GNCH * GCH)], idx_v)

    def body(r, carry):
        for k in range(GSTR):
            pltpu.async_copy(
                xw_hbm.at[idx_v.at[pl.ds(r * GROWS + k * GCH, GCH)]],
                rows_v.at[pl.ds(k * GCH, GCH)], sem)
        pltpu.make_async_copy(xw_hbm.at[pl.ds(0, GROWS)], rows_v, sem).wait()
        pltpu.sync_copy(rows_v, g_hbm.at[pl.ds(e0 + r * GROWS, GROWS)])
        return carry

    lax.fori_loop(0, GRND, body, 0)


def _sc_gather(xw, src):
    return pl.kernel(
        _sc_gather_body,
        out_type=jax.ShapeDtypeStruct((N_EDGES, 16), jnp.float32),
        mesh=plsc.VectorSubcoreMesh(core_axis_name="c", subcore_axis_name="s"),
        compiler_params=pltpu.CompilerParams(use_tc_tiling_on_sc=False, needs_layout_passes=False),
        scratch_types=[
            pltpu.VMEM((GNCH * GCH,), jnp.int32),
            pltpu.VMEM((GROWS, 16), jnp.float32),
            pltpu.SemaphoreType.DMA,
        ],
    )(xw, src)


def _sc_scatter_body(m_hbm, dst_hbm, part_hbm, dst_v, m_v, r_v, sem0, sem1):
    wid = lax.axis_index("s") * NC + lax.axis_index("c")
    chunk = wid // 2
    nh = wid % 2
    base_node = nh * NHALF
    neg = jnp.full((16,), -jnp.inf, jnp.float32)
    iota = lax.broadcasted_iota(jnp.int32, (16,), 0)

    def fire(j, b, sem):
        ebase = chunk * SC_EDGES + j * SCH
        pltpu.async_copy(dst_hbm.at[pl.ds(ebase, SCH)], dst_v.at[b], sem)
        pltpu.async_copy(m_hbm.at[pl.ds(ebase * 16, SCH * 16)], m_v.at[b], sem)

    def drain(b, sem):
        pltpu.make_async_copy(dst_hbm.at[pl.ds(0, SCH)], dst_v.at[b], sem).wait()
        pltpu.make_async_copy(m_hbm.at[pl.ds(0, SCH * 16)], m_v.at[b], sem).wait()

    def process(b):
        def group(g, c2):
            d16 = dst_v[b, pl.ds(g * 16, 16)]
            for i2 in range(16):
                d_b = _lane_bcast(d16, i2)
                local = d_b - base_node
                inb = (local >= 0) & (local < NHALF)
                rowi = jnp.where(inb, local, NHALF)
                fidx = rowi * 16 + iota
                mrow = m_v[b, pl.ds((g * 16 + i2) * 16, 16)]
                cur = plsc.load_gather(r_v, [fidx])
                plsc.store_scatter(r_v, [fidx], jnp.maximum(cur, mrow))
            return c2

        lax.fori_loop(0, SCH // 16, group, 0)

    sems = (sem0, sem1)
    fire(0, 0, sems[0])

    def init(i, carry):
        r_v[pl.ds(i * 16, 16)] = neg
        return carry

    lax.fori_loop(0, NHALF + 1, init, 0)

    def do_pair(j0, carry):
        for b in range(2):
            j = j0 * 2 + b
            drain(b, sems[b])

            @pl.when(j < SNCH - 1)
            def _():
                fire(j + 1, 1 - b, sems[1 - b])

            process(b)
        return carry

    lax.fori_loop(0, (SNCH - 1) // 2, do_pair, 0)
    drain(0, sems[0])
    process(0)
    pltpu.sync_copy(r_v.at[pl.ds(0, NHALF * 16)], part_hbm.at[wid])


def _sc_scatter(m_flat, dst):
    return pl.kernel(
        _sc_scatter_body,
        out_type=jax.ShapeDtypeStruct((NW, NHALF * 16), jnp.float32),
        mesh=plsc.VectorSubcoreMesh(core_axis_name="c", subcore_axis_name="s"),
        compiler_params=pltpu.CompilerParams(use_tc_tiling_on_sc=False, needs_layout_passes=False),
        scratch_types=[
            pltpu.VMEM((2, SCH), jnp.int32),
            pltpu.VMEM((2, SCH * 16), jnp.float32),
            pltpu.VMEM(((NHALF + 1) * 16,), jnp.float32),
            pltpu.SemaphoreType.DMA,
            pltpu.SemaphoreType.DMA,
        ],
    )(m_flat, dst)


def kernel(x, edge_index, edge_attr, W_msg1, b_msg1, W_msg2, b_msg2,
           W_udt1, b_udt1, W_udt2, b_udt2):
    src = edge_index[0]
    dst = edge_index[1]
    eye8 = jnp.eye(8, dtype=jnp.float32)

    x8 = x.reshape(N_NODES // 8, 1024)
    kw = jnp.kron(eye8, W_msg1[16:])
    ku = jnp.kron(eye8, W_udt1[:128])
    k2 = jnp.kron(eye8, W_msg2)
    k1b = jnp.kron(eye8, W_udt1[128:])
    k2u = jnp.kron(eye8, W_udt2)
    bu8 = jnp.tile(b_udt1, 8).reshape(1, 128)
    b28 = jnp.tile(b_msg2, 8).reshape(1, 128)
    b2u8 = jnp.tile(b_udt2, 8).reshape(1, 1024)

    xw8, xu8 = _node_pre(x8, kw, ku, bu8)

    k1 = jnp.kron(eye8, W_msg1[:16])
    b18 = jnp.tile(b_msg1, 8).reshape(1, 128)

    g = _sc_gather(xw8.reshape(N_NODES, 16), src)
    g8 = g.reshape(N_EDGES // 8, 128)
    ea8 = edge_attr.reshape(N_EDGES // 8, 128)
    m8 = _mid(ea8, g8, 0, k1, b18, k2, b28)
    parts = _sc_scatter(m8.reshape(-1), dst)
    p8 = parts.reshape(16, N_NODES // 8, 128)

    out8 = _post(p8, xu8, k1b, k2u, b2u8)
    return out8.reshape(N_NODES, 128)
